# baseline (speedup 1.0000x reference)
import functools

import jax
import jax.numpy as jnp
from jax import lax
from jax.experimental import pallas as pl
from jax.experimental.pallas import tpu as pltpu
from jax.experimental.pallas import tpu_sc as plsc

_NUM_CORES = 2
_NUM_SUBCORES = 16
_NW = _NUM_CORES * _NUM_SUBCORES

_GW = 128
_CH = 256
_NBUF = 4


@functools.partial(jax.jit, static_argnames=("n", "d"))
def _gather_rows(xf, table, n, d):
    per_w = n // _NW
    ng = per_w // _CH
    ks = _CH // _GW

    mesh = plsc.VectorSubcoreMesh(core_axis_name="c", subcore_axis_name="s")

    @functools.partial(
        pl.kernel,
        mesh=mesh,
        compiler_params=pltpu.CompilerParams(use_tc_tiling_on_sc=False),
        out_type=jax.ShapeDtypeStruct((n, d), jnp.float32),
        scratch_types=[
            pltpu.VMEM((per_w,), jnp.int32),
        ]
        + [pltpu.VMEM((_CH, d), jnp.float32)] * _NBUF
        + [pltpu.SemaphoreType.DMA] * (2 * _NBUF),
    )
    def k(x_hbm, table_hbm, out_hbm, idx_v, *bufs_and_sems):
        rows = bufs_and_sems[:_NBUF]
        gsems = bufs_and_sems[_NBUF:2 * _NBUF]
        osems = bufs_and_sems[2 * _NBUF:]

        wid = lax.axis_index("s") * _NUM_CORES + lax.axis_index("c")
        base = wid * per_w
        pltpu.sync_copy(x_hbm.at[pl.ds(base, per_w)], idx_v)

        def fire(c, slot):
            for j in range(ks):
                pltpu.make_async_copy(
                    table_hbm.at[idx_v.at[pl.ds(c * _CH + j * _GW, _GW)]],
                    rows[slot].at[pl.ds(j * _GW, _GW)],
                    gsems[slot],
                ).start()

        def wait_gathers(slot):
            for j in range(ks):
                pltpu.make_async_copy(
                    table_hbm.at[idx_v.at[pl.ds(j * _GW, _GW)]],
                    rows[slot].at[pl.ds(j * _GW, _GW)],
                    gsems[slot],
                ).wait()

        def start_out(c, slot):
            pltpu.make_async_copy(
                rows[slot], out_hbm.at[pl.ds(base + c * _CH, _CH)], osems[slot]
            ).start()

        def wait_out(slot):
            pltpu.make_async_copy(
                rows[slot], out_hbm.at[pl.ds(base, _CH)], osems[slot]
            ).wait()

        fire(0, 0)
        fire(1, 1)

        def round_body(r, carry):
            for b in range(_NBUF):
                c = r * _NBUF + b
                nxt = (b + 2) % _NBUF

                @pl.when(c + 2 < ng)
                def _(c=c, b=b, nxt=nxt):
                    @pl.when(c - 2 >= 0)
                    def _():
                        wait_out(nxt)

                    fire(c + 2, nxt)

                wait_gathers(b)
                start_out(c, b)
            return carry

        lax.fori_loop(0, ng // _NBUF, round_body, 0)

        for slot in range(_NBUF):
            wait_out(slot)

    return k(xf, table)


def kernel(x, table):
    b, h = x.shape
    v, d = table.shape
    n = b * h
    xf = x.reshape(n).astype(jnp.int32)
    out = _gather_rows(xf, table, n, d)
    return out.reshape(b, h, d)


# speedup vs baseline: 1.0003x; 1.0003x over previous
import functools

import jax
import jax.numpy as jnp
from jax import lax
from jax.experimental import pallas as pl
from jax.experimental.pallas import tpu as pltpu
from jax.experimental.pallas import tpu_sc as plsc

_NUM_CORES = 2
_NUM_SUBCORES = 16
_NW = _NUM_CORES * _NUM_SUBCORES

_GW = 128
_CH = 128
_NBUF = 10


@functools.partial(jax.jit, static_argnames=("n", "d"))
def _gather_rows(xf, table, n, d):
    per_w = n // _NW
    ng = per_w // _CH
    ks = _CH // _GW

    mesh = plsc.VectorSubcoreMesh(core_axis_name="c", subcore_axis_name="s")

    @functools.partial(
        pl.kernel,
        mesh=mesh,
        compiler_params=pltpu.CompilerParams(use_tc_tiling_on_sc=False),
        out_type=jax.ShapeDtypeStruct((n, d), jnp.float32),
        scratch_types=[
            pltpu.VMEM((per_w,), jnp.int32),
        ]
        + [pltpu.VMEM((_CH, d), jnp.float32)] * _NBUF
        + [pltpu.SemaphoreType.DMA] * (2 * _NBUF),
    )
    def k(x_hbm, table_hbm, out_hbm, idx_v, *bufs_and_sems):
        rows = bufs_and_sems[:_NBUF]
        gsems = bufs_and_sems[_NBUF:2 * _NBUF]
        osems = bufs_and_sems[2 * _NBUF:]

        wid = lax.axis_index("s") * _NUM_CORES + lax.axis_index("c")
        base = wid * per_w
        pltpu.sync_copy(x_hbm.at[pl.ds(base, per_w)], idx_v)

        def fire(c, slot):
            for j in range(ks):
                pltpu.make_async_copy(
                    table_hbm.at[idx_v.at[pl.ds(c * _CH + j * _GW, _GW)]],
                    rows[slot].at[pl.ds(j * _GW, _GW)],
                    gsems[slot],
                ).start()

        def wait_gathers(slot):
            for j in range(ks):
                pltpu.make_async_copy(
                    table_hbm.at[idx_v.at[pl.ds(j * _GW, _GW)]],
                    rows[slot].at[pl.ds(j * _GW, _GW)],
                    gsems[slot],
                ).wait()

        def start_out(c, slot):
            pltpu.make_async_copy(
                rows[slot], out_hbm.at[pl.ds(base + c * _CH, _CH)], osems[slot]
            ).start()

        def wait_out(slot):
            pltpu.make_async_copy(
                rows[slot], out_hbm.at[pl.ds(base, _CH)], osems[slot]
            ).wait()

        F = _NBUF - 2
        for c0 in range(F):
            fire(c0, c0)

        def round_body(r, carry):
            for b in range(_NBUF):
                c = r * _NBUF + b
                nxt = (b + F) % _NBUF

                @pl.when(c + F < ng)
                def _(c=c, b=b, nxt=nxt):
                    @pl.when(c - 2 >= 0)
                    def _():
                        wait_out(nxt)

                    fire(c + F, nxt)

                wait_gathers(b)
                start_out(c, b)
            return carry

        lax.fori_loop(0, ng // _NBUF, round_body, 0)

        for slot in range(_NBUF):
            wait_out(slot)

    return k(xf, table)


def kernel(x, table):
    b, h = x.shape
    v, d = table.shape
    n = b * h
    xf = x.reshape(n).astype(jnp.int32)
    out = _gather_rows(xf, table, n, d)
    return out.reshape(b, h, d)


# speedup vs baseline: 1.4933x; 1.4928x over previous
import functools

import jax
import jax.numpy as jnp
from jax import lax
from jax.experimental import pallas as pl
from jax.experimental.pallas import tpu as pltpu
from jax.experimental.pallas import tpu_sc as plsc

_NUM_CORES = 2
_NUM_SUBCORES = 16
_NW = _NUM_CORES * _NUM_SUBCORES

_CH = 128
_NBUF = 4


@functools.partial(jax.jit, static_argnames=("n", "d"))
def _gather_rows(xf, table, n, d):
    per_w = n // _NW
    ng = per_w // _CH

    mesh = plsc.VectorSubcoreMesh(core_axis_name="c", subcore_axis_name="s")

    @functools.partial(
        pl.kernel,
        mesh=mesh,
        compiler_params=pltpu.CompilerParams(use_tc_tiling_on_sc=True),
        out_type=jax.ShapeDtypeStruct((n, d), jnp.float32),
        scratch_types=[
            pltpu.VMEM((per_w,), jnp.int32),
        ]
        + [pltpu.VMEM((_CH, d), jnp.float32)] * _NBUF
        + [pltpu.SemaphoreType.DMA] * (2 * _NBUF),
    )
    def k(x_hbm, table_hbm, out_hbm, idx_v, *bufs_and_sems):
        rows = bufs_and_sems[:_NBUF]
        gsems = bufs_and_sems[_NBUF:2 * _NBUF]
        osems = bufs_and_sems[2 * _NBUF:]

        wid = lax.axis_index("s") * _NUM_CORES + lax.axis_index("c")
        base = wid * per_w
        pltpu.sync_copy(x_hbm.at[pl.ds(base, per_w)], idx_v)

        def fire(c, slot):
            def group_body(g, carry):
                vec = idx_v[pl.ds(c * _CH + g * 16, 16)]
                for j in range(16):
                    pltpu.make_async_copy(
                        table_hbm.at[pl.ds(vec[j], 1)],
                        rows[slot].at[pl.ds(g * 16 + j, 1)],
                        gsems[slot],
                    ).start()
                return carry

            lax.fori_loop(0, _CH // 16, group_body, 0)

        def wait_gathers(slot):
            def wait_body(i, carry):
                pltpu.make_async_copy(
                    table_hbm.at[pl.ds(0, 1)],
                    rows[slot].at[pl.ds(0, 1)],
                    gsems[slot],
                ).wait()
                return carry

            lax.fori_loop(0, _CH, wait_body, 0)

        def start_out(c, slot):
            pltpu.make_async_copy(
                rows[slot], out_hbm.at[pl.ds(base + c * _CH, _CH)], osems[slot]
            ).start()

        def wait_out(slot):
            pltpu.make_async_copy(
                rows[slot], out_hbm.at[pl.ds(base, _CH)], osems[slot]
            ).wait()

        F = _NBUF - 2
        for c0 in range(F):
            fire(c0, c0)

        def round_body(r, carry):
            for b in range(_NBUF):
                c = r * _NBUF + b
                nxt = (b + F) % _NBUF

                @pl.when(c + F < ng)
                def _(c=c, b=b, nxt=nxt):
                    @pl.when(c - 2 >= 0)
                    def _():
                        wait_out(nxt)

                    fire(c + F, nxt)

                wait_gathers(b)
                start_out(c, b)
            return carry

        lax.fori_loop(0, ng // _NBUF, round_body, 0)

        for slot in range(_NBUF):
            wait_out(slot)

    return k(xf, table)


def kernel(x, table):
    b, h = x.shape
    v, d = table.shape
    n = b * h
    xf = x.reshape(n).astype(jnp.int32)
    out = _gather_rows(xf, table, n, d)
    return out.reshape(b, h, d)
